# Initial kernel scaffold; baseline (speedup 1.0000x reference)
import jax
import jax.numpy as jnp
from jax.experimental import pallas as pl


def kernel(activations):
    raise NotImplementedError("write your pallas kernel here")


# speedup vs baseline: 1.4904x; 1.4904x over previous
import jax
import jax.numpy as jnp
from jax.experimental import pallas as pl

_S = 256
_K = _S * (_S + 1) // 2


def _off(i: int) -> int:
    return _S * i - i * (i - 1) // 2


def _body(x_ref, o_ref):
    x = x_ref[...]
    for i in range(_S):
        o_ref[:, _off(i):_off(i) + (_S - i)] = x[:, i:i + 1] * x[:, i:]


def kernel(activations):
    B = activations.shape[0]
    BR = 128
    return pl.pallas_call(
        _body,
        grid=(B // BR,),
        in_specs=[pl.BlockSpec((BR, _S), lambda r: (r, 0))],
        out_specs=pl.BlockSpec((BR, _K), lambda r: (r, 0)),
        out_shape=jax.ShapeDtypeStruct((B, _K), jnp.float32),
    )(activations)
